# baseline (speedup 1.0000x reference)
import functools

import jax
import jax.numpy as jnp
from jax import lax
from jax.experimental import pallas as pl
from jax.experimental.pallas import tpu as pltpu
from jax.experimental.pallas import tpu_sc as plsc

VOCAB = 100000
D = 128
B = 4096 * 50
C = 64

_info = plsc.get_sparse_core_info()
NC, NS = _info.num_cores, _info.num_subcores
NW = NC * NS
B_PER_W = B // NW
N_CHUNKS = B_PER_W // C


NBUF = 10


@functools.partial(
    pl.kernel,
    out_type=jax.ShapeDtypeStruct((NW, N_CHUNKS, C, D), jnp.float32),
    mesh=plsc.VectorSubcoreMesh(core_axis_name="c", subcore_axis_name="s"),
    scratch_types=[
        pltpu.VMEM((N_CHUNKS, C), jnp.int32),
        *([pltpu.VMEM((C, D), jnp.float32)] * NBUF),
        *([pltpu.SemaphoreType.DMA] * NBUF),
    ],
)
def _sc_gather(tok_hbm, table_hbm, out_hbm, idx_v, *bufs_and_sems):
    bufs = bufs_and_sems[:NBUF]
    sems = bufs_and_sems[NBUF:]
    wid = lax.axis_index("s") * NC + lax.axis_index("c")
    pltpu.sync_copy(tok_hbm.at[wid], idx_v)

    for b in range(NBUF):
        pltpu.async_copy(table_hbm.at[idx_v.at[b]], bufs[b], sems[b])

    def body(p, carry):
        g = p * NBUF
        for b in range(NBUF):
            j = g + b
            pltpu.make_async_copy(
                table_hbm.at[idx_v.at[j]], bufs[b], sems[b]).wait()
            pltpu.sync_copy(bufs[b], out_hbm.at[wid, j])
            jn = j + NBUF

            @pl.when(jn < N_CHUNKS)
            def _():
                pltpu.async_copy(table_hbm.at[idx_v.at[jn]], bufs[b], sems[b])

        return carry

    lax.fori_loop(0, N_CHUNKS // NBUF, body, 0)


def kernel(recipe_tokens, embedding_table):
    toks = recipe_tokens.astype(jnp.int32).reshape(NW, N_CHUNKS, C)
    out = _sc_gather(toks, embedding_table)
    return out.reshape(50, 4096, D).transpose(1, 0, 2)


# speedup vs baseline: 1.0281x; 1.0281x over previous
import functools

import jax
import jax.numpy as jnp
from jax import lax
from jax.experimental import pallas as pl
from jax.experimental.pallas import tpu as pltpu
from jax.experimental.pallas import tpu_sc as plsc

VOCAB = 100000
D = 128
B = 4096 * 50
C = 64

_info = plsc.get_sparse_core_info()
NC, NS = _info.num_cores, _info.num_subcores
NW = NC * NS
B_PER_W = B // NW
N_CHUNKS = B_PER_W // C


NBUF = 10
G = NBUF // 2


@functools.partial(
    pl.kernel,
    out_type=jax.ShapeDtypeStruct((NW, N_CHUNKS, C, D), jnp.float32),
    mesh=plsc.VectorSubcoreMesh(core_axis_name="c", subcore_axis_name="s"),
    scratch_types=[
        pltpu.VMEM((N_CHUNKS, C), jnp.int32),
        pltpu.VMEM((NBUF, C, D), jnp.float32),
        *([pltpu.SemaphoreType.DMA] * NBUF),
    ],
)
def _sc_gather(tok_hbm, table_hbm, out_hbm, idx_v, buf, *sems):
    wid = lax.axis_index("s") * NC + lax.axis_index("c")
    pltpu.sync_copy(tok_hbm.at[wid], idx_v)

    for b in range(NBUF):
        pltpu.async_copy(table_hbm.at[idx_v.at[b]], buf.at[b], sems[b])

    def body(r, carry):
        base = r * NBUF
        for h in range(2):
            for b in range(G):
                slot = h * G + b
                pltpu.make_async_copy(
                    table_hbm.at[idx_v.at[base + slot]],
                    buf.at[slot], sems[slot]).wait()
            pltpu.sync_copy(buf.at[pl.ds(h * G, G)],
                            out_hbm.at[wid, pl.ds(base + h * G, G)])
            for b in range(G):
                slot = h * G + b
                jn = base + NBUF + slot

                @pl.when(jn < N_CHUNKS)
                def _():
                    pltpu.async_copy(table_hbm.at[idx_v.at[jn]],
                                     buf.at[slot], sems[slot])

        return carry

    lax.fori_loop(0, N_CHUNKS // NBUF, body, 0)


def kernel(recipe_tokens, embedding_table):
    toks = recipe_tokens.astype(jnp.int32).T.reshape(NW, N_CHUNKS, C)
    out = _sc_gather(toks, embedding_table)
    return out.reshape(50, 4096, D).transpose(1, 0, 2)


# speedup vs baseline: 1.0295x; 1.0014x over previous
import functools

import jax
import jax.numpy as jnp
from jax import lax
from jax.experimental import pallas as pl
from jax.experimental.pallas import tpu as pltpu
from jax.experimental.pallas import tpu_sc as plsc

D = 128
B = 4096 * 50
C = 128

_info = plsc.get_sparse_core_info()
NC, NS = _info.num_cores, _info.num_subcores
NW = NC * NS
B_PER_W = B // NW
N_CHUNKS = B_PER_W // C
NBUF = 5


@functools.partial(
    pl.kernel,
    out_type=jax.ShapeDtypeStruct((NW, N_CHUNKS, C, D), jnp.float32),
    mesh=plsc.VectorSubcoreMesh(core_axis_name="c", subcore_axis_name="s"),
    scratch_types=[
        pltpu.VMEM((N_CHUNKS, C), jnp.int32),
        *([pltpu.VMEM((C, D), jnp.float32)] * NBUF),
        *([pltpu.SemaphoreType.DMA] * NBUF),
    ],
)
def _sc_gather(tok_hbm, table_hbm, out_hbm, idx_v, *bufs_and_sems):
    bufs = bufs_and_sems[:NBUF]
    sems = bufs_and_sems[NBUF:]
    wid = lax.axis_index("s") * NC + lax.axis_index("c")
    pltpu.sync_copy(tok_hbm.at[wid], idx_v)

    for b in range(NBUF):
        pltpu.async_copy(table_hbm.at[idx_v.at[b]], bufs[b], sems[b])

    def body(p, carry):
        g = p * NBUF
        for b in range(NBUF):
            j = g + b
            pltpu.make_async_copy(
                table_hbm.at[idx_v.at[j]], bufs[b], sems[b]).wait()
            pltpu.sync_copy(bufs[b], out_hbm.at[wid, j])
            jn = j + NBUF

            @pl.when(jn < N_CHUNKS)
            def _():
                pltpu.async_copy(table_hbm.at[idx_v.at[jn]], bufs[b], sems[b])

        return carry

    lax.fori_loop(0, N_CHUNKS // NBUF, body, 0)


def kernel(recipe_tokens, embedding_table):
    toks = recipe_tokens.astype(jnp.int32).T.reshape(NW, N_CHUNKS, C)
    out = _sc_gather(toks, embedding_table)
    return out.reshape(50, 4096, D).transpose(1, 0, 2)


# speedup vs baseline: 1.0326x; 1.0030x over previous
import functools

import jax
import jax.numpy as jnp
from jax import lax
from jax.experimental import pallas as pl
from jax.experimental.pallas import tpu as pltpu
from jax.experimental.pallas import tpu_sc as plsc

D = 128
B = 4096 * 50
C = 128

_info = plsc.get_sparse_core_info()
NC, NS = _info.num_cores, _info.num_subcores
NW = NC * NS
B_PER_W = B // NW
N_CHUNKS = B_PER_W // C
NBUF = 5


@functools.partial(
    pl.kernel,
    out_type=jax.ShapeDtypeStruct((NW, N_CHUNKS, C, D), jnp.float32),
    mesh=plsc.VectorSubcoreMesh(core_axis_name="c", subcore_axis_name="s"),
    scratch_types=[
        pltpu.VMEM((N_CHUNKS, C), jnp.int32),
        *([pltpu.VMEM((C, D), jnp.float32)] * NBUF),
        *([pltpu.SemaphoreType.DMA] * NBUF),
    ],
)
def _sc_gather(tok_hbm, table_hbm, out_hbm, idx_v, *bufs_and_sems):
    bufs = bufs_and_sems[:NBUF]
    sems = bufs_and_sems[NBUF:]
    wid = lax.axis_index("c") * NS + lax.axis_index("s")
    pltpu.sync_copy(tok_hbm.at[wid], idx_v)

    for b in range(NBUF):
        pltpu.async_copy(table_hbm.at[idx_v.at[b]], bufs[b], sems[b])

    def body(p, carry):
        g = p * NBUF
        for b in range(NBUF):
            j = g + b
            pltpu.make_async_copy(
                table_hbm.at[idx_v.at[j]], bufs[b], sems[b]).wait()
            pltpu.sync_copy(bufs[b], out_hbm.at[wid, j])
            jn = j + NBUF

            @pl.when(jn < N_CHUNKS)
            def _():
                pltpu.async_copy(table_hbm.at[idx_v.at[jn]], bufs[b], sems[b])

        return carry

    lax.fori_loop(0, N_CHUNKS // NBUF, body, 0)


def kernel(recipe_tokens, embedding_table):
    toks = recipe_tokens.astype(jnp.int32).T.reshape(NW, N_CHUNKS, C)
    out = _sc_gather(toks, embedding_table)
    return out.reshape(50, 4096, D).transpose(1, 0, 2)


# speedup vs baseline: 1.0338x; 1.0011x over previous
import functools

import jax
import jax.numpy as jnp
from jax import lax
from jax.experimental import pallas as pl
from jax.experimental.pallas import tpu as pltpu
from jax.experimental.pallas import tpu_sc as plsc

D = 128
B = 4096 * 50
C = 128

_info = plsc.get_sparse_core_info()
NC, NS = _info.num_cores, _info.num_subcores
NW = NC * NS
B_PER_W = B // NW
N_CHUNKS = B_PER_W // C
NBUF = 5


@functools.partial(
    pl.kernel,
    out_type=jax.ShapeDtypeStruct((NW, N_CHUNKS, C, D), jnp.float32),
    mesh=plsc.VectorSubcoreMesh(core_axis_name="c", subcore_axis_name="s"),
    scratch_types=[
        pltpu.VMEM((N_CHUNKS, C), jnp.int32),
        *([pltpu.VMEM((C, D), jnp.float32)] * NBUF),
        *([pltpu.SemaphoreType.DMA] * NBUF),
    ],
    compiler_params=pltpu.CompilerParams(skip_device_barrier=True),
)
def _sc_gather(tok_hbm, table_hbm, out_hbm, idx_v, *bufs_and_sems):
    bufs = bufs_and_sems[:NBUF]
    sems = bufs_and_sems[NBUF:]
    wid = lax.axis_index("c") * NS + lax.axis_index("s")
    pltpu.sync_copy(tok_hbm.at[wid], idx_v)

    for b in range(NBUF):
        pltpu.async_copy(table_hbm.at[idx_v.at[b]], bufs[b], sems[b])

    def body(p, carry):
        g = p * NBUF
        for b in range(NBUF):
            j = g + b
            pltpu.make_async_copy(
                table_hbm.at[idx_v.at[j]], bufs[b], sems[b]).wait()
            pltpu.sync_copy(bufs[b], out_hbm.at[wid, j])
            jn = j + NBUF

            @pl.when(jn < N_CHUNKS)
            def _():
                pltpu.async_copy(table_hbm.at[idx_v.at[jn]], bufs[b], sems[b])

        return carry

    lax.fori_loop(0, N_CHUNKS // NBUF, body, 0)


def kernel(recipe_tokens, embedding_table):
    toks = recipe_tokens.astype(jnp.int32).T.reshape(NW, N_CHUNKS, C)
    out = _sc_gather(toks, embedding_table)
    return out.reshape(50, 4096, D).transpose(1, 0, 2)
